# Initial kernel scaffold; baseline (speedup 1.0000x reference)
import jax
import jax.numpy as jnp
from jax.experimental import pallas as pl


def kernel(t, alphas_bar):
    raise NotImplementedError("write your pallas kernel here")


# speedup vs baseline: 3.3493x; 3.3493x over previous
import functools

import jax
import jax.numpy as jnp
from jax import lax
from jax.experimental import pallas as pl
from jax.experimental.pallas import tpu as pltpu
from jax.experimental.pallas import tpu_sc as plsc

T = 50
TPAD = 64
NSTATS = 14
B = 16384
NC, NS, L = 2, 16, 16
NW = NC * NS
BPW = B // NW
GROUPS = BPW // L


def _rsqrt(v):
    bits = lax.bitcast_convert_type(v, jnp.int32)
    y = lax.bitcast_convert_type(
        jnp.int32(0x5F3759DF) - (bits >> 1), jnp.float32)
    for _ in range(3):
        y = y * (1.5 - 0.5 * v * y * y)
    return y


def _sqrt(v):
    return v * _rsqrt(v)


_mesh = plsc.VectorSubcoreMesh(core_axis_name="c", subcore_axis_name="s")


@functools.partial(
    pl.kernel,
    mesh=_mesh,
    out_type=jax.ShapeDtypeStruct((NSTATS, B), jnp.float32),
    scratch_types=[
        pltpu.VMEM((TPAD,), jnp.float32),
        pltpu.VMEM((BPW,), jnp.int32),
        pltpu.VMEM((NSTATS * TPAD,), jnp.float32),
        pltpu.VMEM((NSTATS * BPW,), jnp.float32),
        pltpu.SemaphoreType.DMA,
    ],
    compiler_params=pltpu.CompilerParams(needs_layout_passes=False),
)
def _sc_lookup(ab_hbm, t_hbm, out_hbm, ab_v, t_v, stats_v, out_v, sem):
    wid = lax.axis_index("s") * NC + lax.axis_index("c")
    base = wid * BPW

    pltpu.sync_copy(ab_hbm, ab_v)
    pltpu.sync_copy(t_hbm.at[pl.ds(base, BPW)], t_v)

    for g in range(TPAD // L):
        gidx = lax.iota(jnp.int32, L) + (g * L)
        x = ab_v[pl.ds(g * L, L)]
        xp = plsc.load_gather(ab_v, [jnp.maximum(gidx - 1, 0)])
        xm = plsc.load_gather(ab_v, [jnp.maximum(gidx, 1)])
        alpha = jnp.where(gidx == 0, x, x / xp)
        beta = 1.0 - alpha
        bbar = 1.0 - x
        sig2 = (1.0 - xm / xp) * (1.0 - xp) / (1.0 - xm)
        sqrt_alpha = _sqrt(alpha)
        sqrt_bbar = _sqrt(bbar)
        rows = (
            x,
            bbar,
            _sqrt(x),
            sqrt_bbar,
            alpha,
            beta,
            sqrt_alpha,
            _sqrt(beta),
            beta * beta,
            sig2,
            _sqrt(sig2),
            1.0 / sqrt_alpha,
            1.0 / sqrt_bbar,
            beta * beta / (2.0 * sig2 * alpha * bbar),
        )
        for r, vec in enumerate(rows):
            stats_v[pl.ds(r * TPAD + g * L, L)] = vec

    @plsc.parallel_loop(jnp.int32(0), jnp.int32(BPW), step=jnp.int32(L),
                        unroll=2)
    def _(off):
        t16 = t_v[pl.ds(off, L)]
        for r in range(NSTATS):
            out_v[pl.ds(off + jnp.int32(r * BPW), L)] = plsc.load_gather(
                stats_v, [t16 + jnp.int32(r * TPAD)])

    for r in range(NSTATS):
        pltpu.sync_copy(out_v.at[pl.ds(r * BPW, BPW)],
                        out_hbm.at[jnp.int32(r), pl.ds(base, BPW)])


def kernel(t, alphas_bar):
    ab32 = jnp.concatenate(
        [alphas_bar.astype(jnp.float32),
         jnp.full((TPAD - T,), 0.5, jnp.float32)])
    t32 = t.astype(jnp.int32)
    return _sc_lookup(ab32, t32)
